# baseline (device time: 36669 ns/iter reference)
import jax
import jax.numpy as jnp
from jax import lax
from jax.experimental import pallas as pl
from jax.experimental.pallas import tpu as pltpu

N_DEV = 32


def kernel(x, w_mat):
    m_per, k = x.shape
    n = w_mat.shape[1]
    n_per = n // N_DEV
    m_total = m_per * N_DEV

    def body(x_ref, w_ref, out_ref, ybf_ref, recv_ref, send_sems, recv_sems):
        me = lax.axis_index("i")

        barrier_sem = pltpu.get_barrier_semaphore()
        for nbr in [lax.rem(me + 1, N_DEV), lax.rem(me + N_DEV - 1, N_DEV)]:
            pl.semaphore_signal(
                barrier_sem, inc=1,
                device_id=(nbr,), device_id_type=pl.DeviceIdType.MESH,
            )
        pl.semaphore_wait(barrier_sem, 2)

        xb = x_ref[:, :].astype(jnp.bfloat16)
        wb = w_ref[:, :].astype(jnp.bfloat16)
        y = jnp.dot(xb, wb, preferred_element_type=jnp.float32)
        y = y * jax.nn.sigmoid(y)
        yb = y.astype(jnp.bfloat16)
        for d in range(N_DEV):
            ybf_ref[d] = yb[:, d * n_per:(d + 1) * n_per]

        rdmas = []
        for j in range(1, N_DEV):
            d = lax.rem(me + j, N_DEV)
            rdma = pltpu.make_async_remote_copy(
                src_ref=ybf_ref.at[d],
                dst_ref=recv_ref.at[j],
                send_sem=send_sems.at[j],
                recv_sem=recv_sems.at[j],
                device_id=(d,),
                device_id_type=pl.DeviceIdType.MESH,
            )
            rdma.start()
            rdmas.append(rdma)

        out_ref[pl.ds(me * m_per, m_per), :] = ybf_ref[me].astype(jnp.float32)

        for j in range(1, N_DEV):
            rdmas[j - 1].wait_recv()
            src = lax.rem(me + N_DEV - j, N_DEV)
            out_ref[pl.ds(src * m_per, m_per), :] = recv_ref[j].astype(jnp.float32)

        for j in range(1, N_DEV):
            rdmas[j - 1].wait_send()

    return pl.pallas_call(
        body,
        out_shape=jax.ShapeDtypeStruct((m_total, n_per), jnp.float32),
        in_specs=[
            pl.BlockSpec(memory_space=pltpu.VMEM),
            pl.BlockSpec(memory_space=pltpu.VMEM),
        ],
        out_specs=pl.BlockSpec(memory_space=pltpu.VMEM),
        scratch_shapes=[
            pltpu.VMEM((N_DEV, m_per, n_per), jnp.bfloat16),
            pltpu.VMEM((N_DEV, m_per, n_per), jnp.bfloat16),
            pltpu.SemaphoreType.DMA((N_DEV,)),
            pltpu.SemaphoreType.DMA((N_DEV,)),
        ],
        compiler_params=pltpu.CompilerParams(
            vmem_limit_bytes=100 * 1024 * 1024,
            collective_id=0,
        ),
    )(x, w_mat)


# device time: 35169 ns/iter; 1.0427x vs baseline; 1.0427x over previous
import jax
import jax.numpy as jnp
from jax import lax
from jax.experimental import pallas as pl
from jax.experimental.pallas import tpu as pltpu

N_DEV = 32
N_CHUNKS = 4


def kernel(x, w_mat):
    m_per, k = x.shape
    n = w_mat.shape[1]
    n_per = n // N_DEV
    m_total = m_per * N_DEV
    n_chunk = n // N_CHUNKS
    d_per_chunk = N_DEV // N_CHUNKS

    def body(x_ref, w_ref, out_ref, ybf_ref, recv_ref, send_sems, recv_sems):
        me = lax.axis_index("i")

        barrier_sem = pltpu.get_barrier_semaphore()
        for nbr in [lax.rem(me + 1, N_DEV), lax.rem(me + N_DEV - 1, N_DEV)]:
            pl.semaphore_signal(
                barrier_sem, inc=1,
                device_id=(nbr,), device_id_type=pl.DeviceIdType.MESH,
            )
        pl.semaphore_wait(barrier_sem, 2)

        xb = x_ref[:, :].astype(jnp.bfloat16)

        rdmas = []
        for c in range(N_CHUNKS):
            wbc = w_ref[:, c * n_chunk:(c + 1) * n_chunk].astype(jnp.bfloat16)
            yc = jnp.dot(xb, wbc, preferred_element_type=jnp.float32)
            yc = yc * jax.nn.sigmoid(yc)
            yc = yc.astype(jnp.bfloat16)
            for b in range(d_per_chunk):
                d = c * d_per_chunk + b
                ybf_ref[d] = yc[:, b * n_per:(b + 1) * n_per]
                rdma = pltpu.make_async_remote_copy(
                    src_ref=ybf_ref.at[d],
                    dst_ref=recv_ref.at[me],
                    send_sem=send_sems.at[d],
                    recv_sem=recv_sems.at[me],
                    device_id=(d,),
                    device_id_type=pl.DeviceIdType.MESH,
                )
                rdmas.append((d, rdma))

                @pl.when(d != me)
                def _start(rdma=rdma):
                    rdma.start()

        out_ref[pl.ds(me * m_per, m_per), :] = ybf_ref[me].astype(jnp.float32)

        for s in range(N_DEV):
            recv = pltpu.make_async_remote_copy(
                src_ref=ybf_ref.at[s],
                dst_ref=recv_ref.at[s],
                send_sem=send_sems.at[s],
                recv_sem=recv_sems.at[s],
                device_id=(s,),
                device_id_type=pl.DeviceIdType.MESH,
            )

            @pl.when(s != me)
            def _drain(recv=recv, s=s):
                recv.wait_recv()
                out_ref[s * m_per:(s + 1) * m_per, :] = recv_ref[s].astype(
                    jnp.float32
                )

        for d, rdma in rdmas:
            @pl.when(d != me)
            def _ws(rdma=rdma):
                rdma.wait_send()

    return pl.pallas_call(
        body,
        out_shape=jax.ShapeDtypeStruct((m_total, n_per), jnp.float32),
        in_specs=[
            pl.BlockSpec(memory_space=pltpu.VMEM),
            pl.BlockSpec(memory_space=pltpu.VMEM),
        ],
        out_specs=pl.BlockSpec(memory_space=pltpu.VMEM),
        scratch_shapes=[
            pltpu.VMEM((N_DEV, m_per, n_per), jnp.bfloat16),
            pltpu.VMEM((N_DEV, m_per, n_per), jnp.bfloat16),
            pltpu.SemaphoreType.DMA((N_DEV,)),
            pltpu.SemaphoreType.DMA((N_DEV,)),
        ],
        compiler_params=pltpu.CompilerParams(
            vmem_limit_bytes=100 * 1024 * 1024,
            collective_id=0,
        ),
    )(x, w_mat)


# device time: 33536 ns/iter; 1.0934x vs baseline; 1.0487x over previous
import jax
import jax.numpy as jnp
from jax import lax
from jax.experimental import pallas as pl
from jax.experimental.pallas import tpu as pltpu

N_DEV = 32
N_CHUNKS = 4


def kernel(x, w_mat):
    m_per, k = x.shape
    n = w_mat.shape[1]
    n_per = n // N_DEV
    m_total = m_per * N_DEV
    n_chunk = n // N_CHUNKS
    d_per_chunk = N_DEV // N_CHUNKS

    def body(x_ref, w_ref, out_ref, ybf_ref, recv_ref, send_sems, recv_sems):
        me = lax.axis_index("i")

        barrier_sem = pltpu.get_barrier_semaphore()
        for nbr in [lax.rem(me + 1, N_DEV), lax.rem(me + N_DEV - 1, N_DEV)]:
            pl.semaphore_signal(
                barrier_sem, inc=1,
                device_id=(nbr,), device_id_type=pl.DeviceIdType.MESH,
            )

        xb = x_ref[:, :].astype(jnp.bfloat16)

        rdmas = []
        for c in range(N_CHUNKS):
            wbc = w_ref[:, c * n_chunk:(c + 1) * n_chunk].astype(jnp.bfloat16)
            yc = jnp.dot(xb, wbc, preferred_element_type=jnp.float32)
            yc = yc * jax.nn.sigmoid(yc)
            yc = yc.astype(jnp.bfloat16)
            for b in range(d_per_chunk):
                d = c * d_per_chunk + b
                ybf_ref[d] = yc[:, b * n_per:(b + 1) * n_per]
            if c == 0:
                pl.semaphore_wait(barrier_sem, 2)
            for b in range(d_per_chunk):
                d = c * d_per_chunk + b
                rdma = pltpu.make_async_remote_copy(
                    src_ref=ybf_ref.at[d],
                    dst_ref=recv_ref.at[me],
                    send_sem=send_sems.at[d],
                    recv_sem=recv_sems.at[me],
                    device_id=(d,),
                    device_id_type=pl.DeviceIdType.MESH,
                )
                rdmas.append((d, rdma))

                @pl.when(d != me)
                def _start(rdma=rdma):
                    rdma.start()

        out_ref[pl.ds(me * m_per, m_per), :] = ybf_ref[me].astype(jnp.float32)

        for s in range(N_DEV):
            recv = pltpu.make_async_remote_copy(
                src_ref=ybf_ref.at[s],
                dst_ref=recv_ref.at[s],
                send_sem=send_sems.at[s],
                recv_sem=recv_sems.at[s],
                device_id=(s,),
                device_id_type=pl.DeviceIdType.MESH,
            )

            @pl.when(s != me)
            def _drain(recv=recv, s=s):
                recv.wait_recv()
                out_ref[s * m_per:(s + 1) * m_per, :] = recv_ref[s].astype(
                    jnp.float32
                )

        for d, rdma in rdmas:
            @pl.when(d != me)
            def _ws(rdma=rdma):
                rdma.wait_send()

    return pl.pallas_call(
        body,
        out_shape=jax.ShapeDtypeStruct((m_total, n_per), jnp.float32),
        in_specs=[
            pl.BlockSpec(memory_space=pltpu.VMEM),
            pl.BlockSpec(memory_space=pltpu.VMEM),
        ],
        out_specs=pl.BlockSpec(memory_space=pltpu.VMEM),
        scratch_shapes=[
            pltpu.VMEM((N_DEV, m_per, n_per), jnp.bfloat16),
            pltpu.VMEM((N_DEV, m_per, n_per), jnp.bfloat16),
            pltpu.SemaphoreType.DMA((N_DEV,)),
            pltpu.SemaphoreType.DMA((N_DEV,)),
        ],
        compiler_params=pltpu.CompilerParams(
            vmem_limit_bytes=100 * 1024 * 1024,
            collective_id=0,
        ),
    )(x, w_mat)
